# baseline (speedup 1.0000x reference)
import functools

import jax
import jax.numpy as jnp
from jax import lax
from jax.experimental import pallas as pl
from jax.experimental.pallas import tpu as pltpu
from jax.experimental.pallas import tpu_sc as plsc

B = 2
S = 2048
D_MODEL = 1024
D_FF = 4096
E = 8
TOPK = 2
CAP_F = 1.25
N_TOK = B * S
CAP = int(CAP_F * N_TOK * TOPK / E)
N_SLOT = E * CAP


BT = 512
N_BLK = N_TOK // BT


def _router_body(x_ref, gwt_ref, sidx0_ref, sidx1_ref, w0_ref, w1_ref,
                 cnt_ref, carry_ref):
    i = pl.program_id(0)

    @pl.when(i == 0)
    def _():
        carry_ref[...] = jnp.zeros_like(carry_ref)

    x = x_ref[...]
    logits = jax.lax.dot_general(
        x, gwt_ref[...], (((1,), (0,)), ((), ())))
    probs = 1.0 / (1.0 + jnp.exp(-logits))

    lane_e = jax.lax.broadcasted_iota(jnp.int32, (BT, E), 1)
    v0 = jnp.max(probs, axis=1, keepdims=True)
    idx0 = jnp.min(jnp.where(probs >= v0, lane_e, E), axis=1, keepdims=True)
    oh0 = lane_e == idx0
    probs_m = jnp.where(oh0, -jnp.inf, probs)
    v1 = jnp.max(probs_m, axis=1, keepdims=True)
    idx1 = jnp.min(jnp.where(probs_m >= v1, lane_e, E), axis=1, keepdims=True)
    oh1 = lane_e == idx1

    denom = v0 + v1 + 1e-6
    wt0 = v0 / denom
    wt1 = v1 / denom

    cnt = (oh0 | oh1).astype(jnp.float32)
    r = jax.lax.broadcasted_iota(jnp.int32, (BT, BT), 0)
    c = jax.lax.broadcasted_iota(jnp.int32, (BT, BT), 1)
    tri = (c < r).astype(jnp.float32)
    excl = carry_ref[...][None, :] + jax.lax.dot_general(
        tri, cnt, (((1,), (0,)), ((), ())))
    rank0_2d = jnp.sum(excl * oh0.astype(jnp.float32), axis=1,
                       keepdims=True).astype(jnp.int32)
    rank1_2d = jnp.sum(excl * oh1.astype(jnp.float32), axis=1,
                       keepdims=True).astype(jnp.int32)
    carry_new = carry_ref[...] + jnp.sum(cnt, axis=0)
    carry_ref[...] = carry_new
    cnt_ref[...] = carry_new.astype(jnp.int32)

    valid0 = rank0_2d < CAP
    valid1 = rank1_2d < CAP
    sidx0_ref[...] = jnp.where(valid0, idx0 * CAP + rank0_2d, N_SLOT)[:, 0]
    sidx1_ref[...] = jnp.where(valid1, idx1 * CAP + rank1_2d, N_SLOT)[:, 0]
    w0_ref[...] = jnp.broadcast_to(jnp.where(valid0, wt0, 0.0), (BT, 16))
    w1_ref[...] = jnp.broadcast_to(jnp.where(valid1, wt1, 0.0), (BT, 16))


def _router(x2d, gwt):
    return pl.pallas_call(
        _router_body,
        grid=(N_BLK,),
        in_specs=[
            pl.BlockSpec((BT, D_MODEL), lambda i: (i, 0)),
            pl.BlockSpec((D_MODEL, E), lambda i: (0, 0)),
        ],
        out_specs=[
            pl.BlockSpec((BT,), lambda i: (i,)),
            pl.BlockSpec((BT,), lambda i: (i,)),
            pl.BlockSpec((BT, 16), lambda i: (i, 0)),
            pl.BlockSpec((BT, 16), lambda i: (i, 0)),
            pl.BlockSpec((E,), lambda i: (0,)),
        ],
        out_shape=[
            jax.ShapeDtypeStruct((N_TOK,), jnp.int32),
            jax.ShapeDtypeStruct((N_TOK,), jnp.int32),
            jax.ShapeDtypeStruct((N_TOK, 16), jnp.float32),
            jax.ShapeDtypeStruct((N_TOK, 16), jnp.float32),
            jax.ShapeDtypeStruct((E,), jnp.int32),
        ],
        scratch_shapes=[pltpu.VMEM((E,), jnp.float32)],
    )(x2d, gwt)


NW = 32
TW = N_TOK // NW
CH_B = 64


def _scatter_body(x_hbm, i0_hbm, i1_hbm, out_hbm, i0_v, i1_v, rows_v,
                  sem0, sem1):
    wid = lax.axis_index("s") * 2 + lax.axis_index("c")
    base = wid * TW
    for ci in range(TW // CH_B):
        tb = base + ci * CH_B
        pltpu.sync_copy(i0_hbm.at[pl.ds(tb, CH_B)], i0_v)
        pltpu.sync_copy(i1_hbm.at[pl.ds(tb, CH_B)], i1_v)
        pltpu.sync_copy(x_hbm.at[pl.ds(tb, CH_B)], rows_v)
        cp0 = pltpu.async_copy(rows_v, out_hbm.at[i0_v], sem0)
        cp1 = pltpu.async_copy(rows_v, out_hbm.at[i1_v], sem1)
        cp0.wait()
        cp1.wait()


@functools.lru_cache(maxsize=None)
def _make_scatter():
    return pl.kernel(
        _scatter_body,
        out_type=jax.ShapeDtypeStruct((N_SLOT + 1, D_MODEL), jnp.float32),
        mesh=plsc.VectorSubcoreMesh(core_axis_name="c", subcore_axis_name="s"),
        scratch_types=[
            pltpu.VMEM((CH_B,), jnp.int32),
            pltpu.VMEM((CH_B,), jnp.int32),
            pltpu.VMEM((CH_B, D_MODEL), jnp.float32),
            pltpu.SemaphoreType.DMA,
            pltpu.SemaphoreType.DMA,
        ],
    )


BF = 1024
NF_BLK = D_FF // BF


BSUB = 256
NSUB = CAP // BSUB


def _ffn_body(cnt_ref, x_ref, w1_ref, w2_ref, out_ref):
    e = pl.program_id(0)
    f = pl.program_id(1)
    cnt = cnt_ref[e]
    for c in range(NSUB):
        @pl.when(c * BSUB < cnt)
        def _():
            sl = pl.ds(c * BSUB, BSUB)
            h = jax.lax.dot_general(
                x_ref[sl, :], w1_ref[0], (((1,), (0,)), ((), ())))
            h = 0.5 * h * (1.0 + jax.lax.erf(h * 0.7071067811865476))
            part = jax.lax.dot_general(
                h, w2_ref[0], (((1,), (0,)), ((), ())))

            @pl.when(f == 0)
            def _():
                out_ref[sl, :] = part

            @pl.when(f > 0)
            def _():
                out_ref[sl, :] = out_ref[sl, :] + part


def _ffn(counts, padded, w1, w2):
    return pl.pallas_call(
        _ffn_body,
        grid_spec=pltpu.PrefetchScalarGridSpec(
            num_scalar_prefetch=1,
            grid=(E, NF_BLK),
            in_specs=[
                pl.BlockSpec((CAP, D_MODEL), lambda e, f, cnt: (e, 0)),
                pl.BlockSpec((1, D_MODEL, BF), lambda e, f, cnt: (e, 0, f)),
                pl.BlockSpec((1, BF, D_MODEL), lambda e, f, cnt: (e, f, 0)),
            ],
            out_specs=pl.BlockSpec((CAP, D_MODEL), lambda e, f, cnt: (e, 0)),
        ),
        out_shape=jax.ShapeDtypeStruct((N_SLOT, D_MODEL), jnp.float32),
    )(counts, padded, w1, w2)


CH_D = 32


def _gather_body(ffn_hbm, i0_hbm, i1_hbm, w0_hbm, w1_hbm, out_hbm,
                 i0_v, i1_v, w0_v, w1_v, r0_v, r1_v, sem0, sem1):
    wid = lax.axis_index("s") * 2 + lax.axis_index("c")
    base = wid * TW
    for ci in range(TW // CH_D):
        tb = base + ci * CH_D
        pltpu.sync_copy(i0_hbm.at[pl.ds(tb, CH_D)], i0_v)
        pltpu.sync_copy(i1_hbm.at[pl.ds(tb, CH_D)], i1_v)
        for j in range(CH_D // 16):
            sl = pl.ds(j * 16, 16)
            i0_v[sl] = jnp.minimum(i0_v[sl], N_SLOT - 1)
            i1_v[sl] = jnp.minimum(i1_v[sl], N_SLOT - 1)
        cp0 = pltpu.async_copy(ffn_hbm.at[i0_v], r0_v, sem0)
        cp1 = pltpu.async_copy(ffn_hbm.at[i1_v], r1_v, sem1)
        pltpu.sync_copy(w0_hbm.at[pl.ds(tb, CH_D)], w0_v)
        pltpu.sync_copy(w1_hbm.at[pl.ds(tb, CH_D)], w1_v)
        cp0.wait()
        cp1.wait()

        def tok_loop(t, _):
            wr0 = w0_v[t]
            wr1 = w1_v[t]
            zero = jnp.zeros((16,), jnp.float32)

            def lane_loop(l, _):
                sl = pl.ds(l * 16, 16)
                a = jnp.where(wr0 > 0.0, r0_v[t, sl] * wr0, zero)
                b = jnp.where(wr1 > 0.0, r1_v[t, sl] * wr1, zero)
                r0_v[t, sl] = a + b
                return 0

            return lax.fori_loop(0, D_MODEL // 16, lane_loop, 0)

        lax.fori_loop(0, CH_D, tok_loop, 0)
        pltpu.sync_copy(r0_v, out_hbm.at[pl.ds(tb, CH_D)])


@functools.lru_cache(maxsize=None)
def _make_gather():
    return pl.kernel(
        _gather_body,
        out_type=jax.ShapeDtypeStruct((N_TOK, D_MODEL), jnp.float32),
        mesh=plsc.VectorSubcoreMesh(core_axis_name="c", subcore_axis_name="s"),
        scratch_types=[
            pltpu.VMEM((CH_D,), jnp.int32),
            pltpu.VMEM((CH_D,), jnp.int32),
            pltpu.VMEM((CH_D, 16), jnp.float32),
            pltpu.VMEM((CH_D, 16), jnp.float32),
            pltpu.VMEM((CH_D, D_MODEL), jnp.float32),
            pltpu.VMEM((CH_D, D_MODEL), jnp.float32),
            pltpu.SemaphoreType.DMA,
            pltpu.SemaphoreType.DMA,
        ],
    )


def kernel(x, gate_w, w1, w2):
    x2d = x.reshape(N_TOK, D_MODEL)
    gwt = gate_w.T
    sidx0, sidx1, w0e, w1e, counts = _router(x2d, gwt)
    padded = _make_scatter()(x2d, sidx0, sidx1)
    ffn = _ffn(counts, padded, w1, w2)
    out = _make_gather()(ffn, sidx0, sidx1, w0e, w1e)
    return out.reshape(B, S, D_MODEL)


# speedup vs baseline: 1.1107x; 1.1107x over previous
import functools

import jax
import jax.numpy as jnp
from jax import lax
from jax.experimental import pallas as pl
from jax.experimental.pallas import tpu as pltpu
from jax.experimental.pallas import tpu_sc as plsc

B = 2
S = 2048
D_MODEL = 1024
D_FF = 4096
E = 8
TOPK = 2
CAP_F = 1.25
N_TOK = B * S
CAP = int(CAP_F * N_TOK * TOPK / E)
N_SLOT = E * CAP


BT = 512
N_BLK = N_TOK // BT


def _router_body(x_ref, gwt_ref, sidx0_ref, sidx1_ref, w0_ref, w1_ref,
                 cnt_ref, carry_ref):
    i = pl.program_id(0)

    @pl.when(i == 0)
    def _():
        carry_ref[...] = jnp.zeros_like(carry_ref)

    x = x_ref[...]
    logits = jax.lax.dot_general(
        x, gwt_ref[...], (((1,), (0,)), ((), ())))
    probs = 1.0 / (1.0 + jnp.exp(-logits))

    lane_e = jax.lax.broadcasted_iota(jnp.int32, (BT, E), 1)
    v0 = jnp.max(probs, axis=1, keepdims=True)
    idx0 = jnp.min(jnp.where(probs >= v0, lane_e, E), axis=1, keepdims=True)
    oh0 = lane_e == idx0
    probs_m = jnp.where(oh0, -jnp.inf, probs)
    v1 = jnp.max(probs_m, axis=1, keepdims=True)
    idx1 = jnp.min(jnp.where(probs_m >= v1, lane_e, E), axis=1, keepdims=True)
    oh1 = lane_e == idx1

    denom = v0 + v1 + 1e-6
    wt0 = v0 / denom
    wt1 = v1 / denom

    cnt = (oh0 | oh1).astype(jnp.float32)
    r = jax.lax.broadcasted_iota(jnp.int32, (BT, BT), 0)
    c = jax.lax.broadcasted_iota(jnp.int32, (BT, BT), 1)
    tri = (c < r).astype(jnp.float32)
    excl = carry_ref[...][None, :] + jax.lax.dot_general(
        tri, cnt, (((1,), (0,)), ((), ())))
    rank0_2d = jnp.sum(excl * oh0.astype(jnp.float32), axis=1,
                       keepdims=True).astype(jnp.int32)
    rank1_2d = jnp.sum(excl * oh1.astype(jnp.float32), axis=1,
                       keepdims=True).astype(jnp.int32)
    carry_new = carry_ref[...] + jnp.sum(cnt, axis=0)
    carry_ref[...] = carry_new
    cnt_ref[...] = carry_new.astype(jnp.int32)

    valid0 = rank0_2d < CAP
    valid1 = rank1_2d < CAP
    sidx0_ref[...] = jnp.where(valid0, idx0 * CAP + rank0_2d, N_SLOT)[:, 0]
    sidx1_ref[...] = jnp.where(valid1, idx1 * CAP + rank1_2d, N_SLOT)[:, 0]
    w0_ref[...] = jnp.broadcast_to(jnp.where(valid0, wt0, 0.0), (BT, 16))
    w1_ref[...] = jnp.broadcast_to(jnp.where(valid1, wt1, 0.0), (BT, 16))


def _router(x2d, gwt):
    return pl.pallas_call(
        _router_body,
        grid=(N_BLK,),
        in_specs=[
            pl.BlockSpec((BT, D_MODEL), lambda i: (i, 0)),
            pl.BlockSpec((D_MODEL, E), lambda i: (0, 0)),
        ],
        out_specs=[
            pl.BlockSpec((BT,), lambda i: (i,)),
            pl.BlockSpec((BT,), lambda i: (i,)),
            pl.BlockSpec((BT, 16), lambda i: (i, 0)),
            pl.BlockSpec((BT, 16), lambda i: (i, 0)),
            pl.BlockSpec((E,), lambda i: (0,)),
        ],
        out_shape=[
            jax.ShapeDtypeStruct((N_TOK,), jnp.int32),
            jax.ShapeDtypeStruct((N_TOK,), jnp.int32),
            jax.ShapeDtypeStruct((N_TOK, 16), jnp.float32),
            jax.ShapeDtypeStruct((N_TOK, 16), jnp.float32),
            jax.ShapeDtypeStruct((E,), jnp.int32),
        ],
        scratch_shapes=[pltpu.VMEM((E,), jnp.float32)],
    )(x2d, gwt)


NW = 32
TW = N_TOK // NW
CH_B = 64


def _scatter_body(x_hbm, i0_hbm, i1_hbm, out_hbm, i0_v, i1_v, rows_v,
                  sem0, sem1):
    wid = lax.axis_index("s") * 2 + lax.axis_index("c")
    base = wid * TW
    for ci in range(TW // CH_B):
        tb = base + ci * CH_B
        pltpu.sync_copy(i0_hbm.at[pl.ds(tb, CH_B)], i0_v)
        pltpu.sync_copy(i1_hbm.at[pl.ds(tb, CH_B)], i1_v)
        pltpu.sync_copy(x_hbm.at[pl.ds(tb, CH_B)], rows_v)
        cp0 = pltpu.async_copy(rows_v, out_hbm.at[i0_v], sem0)
        cp1 = pltpu.async_copy(rows_v, out_hbm.at[i1_v], sem1)
        cp0.wait()
        cp1.wait()


@functools.lru_cache(maxsize=None)
def _make_scatter():
    return pl.kernel(
        _scatter_body,
        out_type=jax.ShapeDtypeStruct((N_SLOT + 1, D_MODEL), jnp.float32),
        mesh=plsc.VectorSubcoreMesh(core_axis_name="c", subcore_axis_name="s"),
        scratch_types=[
            pltpu.VMEM((CH_B,), jnp.int32),
            pltpu.VMEM((CH_B,), jnp.int32),
            pltpu.VMEM((CH_B, D_MODEL), jnp.float32),
            pltpu.SemaphoreType.DMA,
            pltpu.SemaphoreType.DMA,
        ],
    )


BF = 1024
NF_BLK = D_FF // BF


BSUB = 256
NSUB = CAP // BSUB


def _ffn_body(cnt_ref, x_ref, w1_ref, w2_ref, out_ref):
    e = pl.program_id(0)
    f = pl.program_id(1)
    cnt = cnt_ref[e]
    for c in range(NSUB):
        @pl.when(c * BSUB < cnt)
        def _():
            sl = pl.ds(c * BSUB, BSUB)
            h = jax.lax.dot_general(
                x_ref[sl, :], w1_ref[0], (((1,), (0,)), ((), ())))
            h = 0.5 * h * (1.0 + jax.lax.erf(h * 0.7071067811865476))
            part = jax.lax.dot_general(
                h, w2_ref[0], (((1,), (0,)), ((), ())))

            @pl.when(f == 0)
            def _():
                out_ref[sl, :] = part

            @pl.when(f > 0)
            def _():
                out_ref[sl, :] = out_ref[sl, :] + part


def _ffn(counts, padded, w1, w2):
    return pl.pallas_call(
        _ffn_body,
        grid_spec=pltpu.PrefetchScalarGridSpec(
            num_scalar_prefetch=1,
            grid=(E, NF_BLK),
            in_specs=[
                pl.BlockSpec((CAP, D_MODEL), lambda e, f, cnt: (e, 0)),
                pl.BlockSpec((1, D_MODEL, BF), lambda e, f, cnt: (e, 0, f)),
                pl.BlockSpec((1, BF, D_MODEL), lambda e, f, cnt: (e, f, 0)),
            ],
            out_specs=pl.BlockSpec((CAP, D_MODEL), lambda e, f, cnt: (e, 0)),
        ),
        out_shape=jax.ShapeDtypeStruct((N_SLOT, D_MODEL), jnp.float32),
    )(counts, padded, w1, w2)


CH_D = 32


def _gather_body(ffn_hbm, i0_hbm, i1_hbm, w0_hbm, w1_hbm, out_hbm,
                 i0_v, i1_v, w0_v, w1_v, r0_v, r1_v, sem0, sem1):
    wid = lax.axis_index("s") * 2 + lax.axis_index("c")
    base = wid * TW
    for ci in range(TW // CH_D):
        tb = base + ci * CH_D
        pltpu.sync_copy(i0_hbm.at[pl.ds(tb, CH_D)], i0_v)
        pltpu.sync_copy(i1_hbm.at[pl.ds(tb, CH_D)], i1_v)
        for j in range(CH_D // 16):
            sl = pl.ds(j * 16, 16)
            i0_v[sl] = jnp.minimum(i0_v[sl], N_SLOT - 1)
            i1_v[sl] = jnp.minimum(i1_v[sl], N_SLOT - 1)
        cp0 = pltpu.async_copy(ffn_hbm.at[i0_v], r0_v, sem0)
        cp1 = pltpu.async_copy(ffn_hbm.at[i1_v], r1_v, sem1)
        pltpu.sync_copy(w0_hbm.at[pl.ds(tb, CH_D)], w0_v)
        pltpu.sync_copy(w1_hbm.at[pl.ds(tb, CH_D)], w1_v)
        cp0.wait()
        cp1.wait()

        def tok_loop(t, _):
            wr0 = w0_v[t]
            wr1 = w1_v[t]
            zero = jnp.zeros((16,), jnp.float32)
            for l in range(D_MODEL // 16):
                sl = pl.ds(l * 16, 16)
                a = jnp.where(wr0 > 0.0, r0_v[t, sl] * wr0, zero)
                b = jnp.where(wr1 > 0.0, r1_v[t, sl] * wr1, zero)
                r0_v[t, sl] = a + b
            return 0

        lax.fori_loop(0, CH_D, tok_loop, 0)
        pltpu.sync_copy(r0_v, out_hbm.at[pl.ds(tb, CH_D)])


@functools.lru_cache(maxsize=None)
def _make_gather():
    return pl.kernel(
        _gather_body,
        out_type=jax.ShapeDtypeStruct((N_TOK, D_MODEL), jnp.float32),
        mesh=plsc.VectorSubcoreMesh(core_axis_name="c", subcore_axis_name="s"),
        scratch_types=[
            pltpu.VMEM((CH_D,), jnp.int32),
            pltpu.VMEM((CH_D,), jnp.int32),
            pltpu.VMEM((CH_D, 16), jnp.float32),
            pltpu.VMEM((CH_D, 16), jnp.float32),
            pltpu.VMEM((CH_D, D_MODEL), jnp.float32),
            pltpu.VMEM((CH_D, D_MODEL), jnp.float32),
            pltpu.SemaphoreType.DMA,
            pltpu.SemaphoreType.DMA,
        ],
    )


def kernel(x, gate_w, w1, w2):
    x2d = x.reshape(N_TOK, D_MODEL)
    gwt = gate_w.T
    sidx0, sidx1, w0e, w1e, counts = _router(x2d, gwt)
    padded = _make_scatter()(x2d, sidx0, sidx1)
    ffn = _ffn(counts, padded, w1, w2)
    out = _make_gather()(ffn, sidx0, sidx1, w0e, w1e)
    return out.reshape(B, S, D_MODEL)
